# baseline (speedup 1.0000x reference)
import jax
import jax.numpy as jnp
from jax.experimental import pallas as pl

N, E, A, TK = 16384, 64, 64, 32
R, C = 128, 128


def _body(g_ref, s_ref, wq_ref, bq_ref, wk_ref, bk_ref, wv_ref, bv_ref,
          wp1_ref, bp1_ref, wp2_ref, bp2_ref, wf1_ref, bf1_ref, wf2_ref,
          bf2_ref, out_ref, aw_ref, idx_ref, tw_ref):
    g = g_ref[...]
    s = s_ref[...]

    q = jnp.dot(s, wq_ref[...], preferred_element_type=jnp.float32) + bq_ref[...]
    k = jnp.dot(g, wk_ref[...], preferred_element_type=jnp.float32) + bk_ref[...]
    score = jnp.sum(q * k, axis=1) * (1.0 / (A ** 0.5))
    sc2d = score.reshape(R, C)

    mx = jnp.max(sc2d)
    e = jnp.exp(sc2d - mx)
    aw = e / jnp.sum(e)
    aw_ref[...] = aw

    row_i = jax.lax.broadcasted_iota(jnp.int32, (R, C), 0)
    col_i = jax.lax.broadcasted_iota(jnp.int32, (R, C), 1)
    flat_i = row_i * C + col_i
    lane32 = jax.lax.broadcasted_iota(jnp.int32, (1, TK), 1)

    def step(i, carry):
        a, idx_acc, w_acc, ws = carry
        m = jnp.max(a)
        fidx = jnp.min(jnp.where(a == m, flat_i, jnp.int32(1 << 30)))
        a = jnp.where(flat_i == fidx, jnp.float32(-1.0), a)
        idx_acc = jnp.where(lane32 == i, fidx, idx_acc)
        w_acc = jnp.where(lane32 == i, m, w_acc)
        ws = ws + m * s_ref[pl.ds(fidx, 1), :]
        return a, idx_acc, w_acc, ws

    init = (aw, jnp.zeros((1, TK), jnp.int32), jnp.zeros((1, TK), jnp.float32),
            jnp.zeros((1, E), jnp.float32))
    _, idx_acc, w_acc, ws = jax.lax.fori_loop(0, TK, step, init)
    idx_ref[...] = idx_acc
    tw_ref[...] = w_acc

    wsum = jnp.sum(w_acc)
    pc0 = jnp.dot(ws, wv_ref[...], preferred_element_type=jnp.float32) \
        + wsum * bv_ref[...]
    h = jnp.maximum(
        jnp.dot(pc0, wp1_ref[...], preferred_element_type=jnp.float32)
        + bp1_ref[...], 0.0)
    pc = jnp.dot(h, wp2_ref[...], preferred_element_type=jnp.float32) \
        + bp2_ref[...]

    c_row = jnp.dot(pc, wf1_ref[E:, :], preferred_element_type=jnp.float32) \
        + bf1_ref[...]
    h2 = jnp.maximum(
        jnp.dot(g, wf1_ref[:E, :], preferred_element_type=jnp.float32)
        + c_row, 0.0)
    out_ref[...] = jnp.dot(h2, wf2_ref[...], preferred_element_type=jnp.float32) \
        + bf2_ref[...]


def kernel(graph_repr, substructure_repr, Wq, bq, Wk, bk, Wv, bv,
           Wp1, bp1, Wp2, bp2, Wf1, bf1, Wf2, bf2):
    out, aw, idx, tw = pl.pallas_call(
        _body,
        out_shape=[
            jax.ShapeDtypeStruct((N, E), jnp.float32),
            jax.ShapeDtypeStruct((R, C), jnp.float32),
            jax.ShapeDtypeStruct((1, TK), jnp.int32),
            jax.ShapeDtypeStruct((1, TK), jnp.float32),
        ],
    )(graph_repr, substructure_repr,
      Wq, bq.reshape(1, A), Wk, bk.reshape(1, A), Wv, bv.reshape(1, E),
      Wp1, bp1.reshape(1, A), Wp2, bp2.reshape(1, E),
      Wf1, bf1.reshape(1, A), Wf2, bf2.reshape(1, E))
    return out, aw.reshape(N), idx.reshape(TK), tw.reshape(TK)
